# baseline (speedup 1.0000x reference)
import functools

import jax
import jax.numpy as jnp
from jax import lax
from jax.experimental import pallas as pl
from jax.experimental.pallas import tpu as pltpu
from jax.experimental.pallas import tpu_sc as plsc

_N = 10000
_E = 320000
_D = 128
_H = 8
_DK = 16
_FF = 512

_PREC = lax.Precision.HIGHEST

_NW = 32
_EPW = _E // _NW
_B = 40
_NBLK = _EPW // _B
_CW = _D + _DK
_NPAD = 10240
_RPS = _NPAD // 16


def _qkv_body(x_ref, wq_ref, bq_ref, wk_ref, wv_ref, q_ref, k_ref, v_ref):
    x = x_ref[...]
    q_ref[...] = jnp.dot(x, wq_ref[...], precision=_PREC) + bq_ref[...]
    k_ref[...] = jnp.dot(x, wk_ref[...], precision=_PREC)
    v_ref[...] = jnp.dot(x, wv_ref[...], precision=_PREC)


def _qkv(x, wq, bq, wk, wv):
    blk = 2000
    f = pl.pallas_call(
        _qkv_body,
        grid=(_N // blk,),
        in_specs=[
            pl.BlockSpec((blk, _D), lambda i: (i, 0)),
            pl.BlockSpec((_D, _D), lambda i: (0, 0)),
            pl.BlockSpec((1, _D), lambda i: (0, 0)),
            pl.BlockSpec((_D, _D), lambda i: (0, 0)),
            pl.BlockSpec((_D, _D), lambda i: (0, 0)),
        ],
        out_specs=[pl.BlockSpec((blk, _D), lambda i: (i, 0))] * 3,
        out_shape=[jax.ShapeDtypeStruct((_N, _D), jnp.float32)] * 3,
    )
    return f(x, wq, bq.reshape(1, _D), wk, wv)


def _ln(x, g, b):
    mu = jnp.mean(x, axis=-1, keepdims=True)
    var = jnp.mean((x - mu) ** 2, axis=-1, keepdims=True)
    return (x - mu) / jnp.sqrt(var + 1e-5) * g + b


def _post_body(x_ref, m0_ref, m1_ref, z0_ref, z1_ref,
               wo_ref, bo_ref, g1_ref, bb1_ref,
               w1_ref, bf1_ref, w2_ref, bf2_ref, g2_ref, bb2_ref, out_ref):
    wv = m0_ref[...] + m1_ref[...]
    z8 = z0_ref[:, :_H] + z1_ref[:, :_H]
    sel = (lax.broadcasted_iota(jnp.int32, (_H, _D), 1) // _DK
           == lax.broadcasted_iota(jnp.int32, (_H, _D), 0)).astype(jnp.float32)
    zf = jnp.dot(z8, sel, precision=_PREC)
    o = wv / zf
    a = x_ref[...] + jnp.dot(o, wo_ref[...], precision=_PREC) + bo_ref[...]
    a = _ln(a, g1_ref[...], bb1_ref[...])
    hid = jnp.maximum(jnp.dot(a, w1_ref[...], precision=_PREC) + bf1_ref[...], 0.0)
    f = jnp.dot(hid, w2_ref[...], precision=_PREC) + bf2_ref[...]
    out_ref[...] = _ln(a + f, g2_ref[...], bb2_ref[...])


def _post(x, m0, m1, z0, z1, p):
    blk = 2000
    f = pl.pallas_call(
        _post_body,
        grid=(_N // blk,),
        in_specs=[
            pl.BlockSpec((blk, _D), lambda i: (i, 0)),
            pl.BlockSpec((blk, _D), lambda i: (i, 0)),
            pl.BlockSpec((blk, _D), lambda i: (i, 0)),
            pl.BlockSpec((blk, _D), lambda i: (i, 0)),
            pl.BlockSpec((blk, _D), lambda i: (i, 0)),
            pl.BlockSpec((_D, _D), lambda i: (0, 0)),
            pl.BlockSpec((1, _D), lambda i: (0, 0)),
            pl.BlockSpec((1, _D), lambda i: (0, 0)),
            pl.BlockSpec((1, _D), lambda i: (0, 0)),
            pl.BlockSpec((_D, _FF), lambda i: (0, 0)),
            pl.BlockSpec((1, _FF), lambda i: (0, 0)),
            pl.BlockSpec((_FF, _D), lambda i: (0, 0)),
            pl.BlockSpec((1, _D), lambda i: (0, 0)),
            pl.BlockSpec((1, _D), lambda i: (0, 0)),
            pl.BlockSpec((1, _D), lambda i: (0, 0)),
        ],
        out_specs=pl.BlockSpec((blk, _D), lambda i: (i, 0)),
        out_shape=jax.ShapeDtypeStruct((_N, _D), jnp.float32),
    )
    return f(x, m0, m1, z0, z1, p["Wo"], p["bo"].reshape(1, _D),
             p["ln1_g"].reshape(1, _D), p["ln1_b"].reshape(1, _D),
             p["W1"], p["b1"].reshape(1, _FF), p["W2"],
             p["b2"].reshape(1, _D), p["ln2_g"].reshape(1, _D),
             p["ln2_b"].reshape(1, _D))


def _xlane(t, pm):
    return lax.gather(
        t, pm[:, None],
        lax.GatherDimensionNumbers(offset_dims=(), collapsed_slice_dims=(0,),
                                   start_index_map=(0,)),
        slice_sizes=(1,), mode=lax.GatherScatterMode.PROMISE_IN_BOUNDS)


_CB = 10
_CE = _CB * _B
_NCH = _EPW // _CE


def _edge_sc(q, k, v, src, dst1, rel, re):
    mesh = plsc.VectorSubcoreMesh(core_axis_name="c", subcore_axis_name="s")

    @functools.partial(
        pl.kernel,
        mesh=mesh,
        out_type=[
            jax.ShapeDtypeStruct((2, _NPAD, _D), jnp.float32),
            jax.ShapeDtypeStruct((2, _NPAD, _D), jnp.float32),
            jax.ShapeDtypeStruct((_E, _DK), jnp.float32),
        ],
        scratch_types=[
            pltpu.VMEM((_CE,), jnp.int32),
            pltpu.VMEM((_CE,), jnp.int32),
            pltpu.VMEM((_B,), jnp.int32),
            pltpu.VMEM((_B,), jnp.int32),
            pltpu.VMEM((_CE + 16,), jnp.int32),
            pltpu.VMEM((_B, _D), jnp.float32),
            pltpu.VMEM((_B, _D), jnp.float32),
            pltpu.VMEM((_B, _D), jnp.float32),
            pltpu.VMEM((_B, _D), jnp.float32),
            pltpu.VMEM((_B, _DK), jnp.float32),
            pltpu.VMEM((_B, _DK), jnp.float32),
            pltpu.VMEM((100, _DK), jnp.float32),
            pltpu.VMEM_SHARED((_NPAD, _D), jnp.float32),
            pltpu.SemaphoreType.DMA,
            pltpu.SemaphoreType.DMA,
            pltpu.SemaphoreType.DMA,
            pltpu.SemaphoreType.DMA,
            pltpu.SemaphoreType.DMA,
            pltpu.SemaphoreType.DMA,
            pltpu.SemaphoreType.DMA,
            pltpu.SemaphoreType.DMA,
        ],
    )
    def ek(q_hbm, k_hbm, v_hbm, src_hbm, dst1_hbm, rel_hbm, re_hbm,
           zout_hbm, mout_hbm, sc_hbm,
           src_c, dst_f, dst_s0, dst_s1, rel_c, a0, a1, b0, b1,
           s0, s1, re_v,
           acc, sga0, sga1, sgb0, sgb1, ssc0, ssc1, ssp0, ssp1):
        cid = lax.axis_index("c")
        sid = lax.axis_index("s")
        wid = cid * 16 + sid
        wbase = wid * _EPW
        wrow0 = wid * (_EPW // _B)

        ab = [(a0, sga0), (a1, sga1)]
        bb = [(b0, sgb0), (b1, sgb1)]
        wb = [b0, b1]
        sb = [s0, s1]
        db = [dst_s0, dst_s1]
        scs = [ssc0, ssc1]
        sps = [ssp0, ssp1]

        zvec = jnp.zeros((16,), jnp.float32)
        lane = lax.iota(jnp.int32, 16)
        perms = [lane ^ (1 << t) for t in range(4)]
        hsplat = [jnp.full((16,), h, jnp.int32) for h in range(_H)]
        lane8 = lane < _H

        def zero_fill(r, carry):
            for cc in range(_D // 16):
                b0[r, pl.ds(cc * 16, 16)] = zvec
            return carry

        def zero_acc():
            lax.fori_loop(0, _B, zero_fill, 0)
            for t in range(_RPS // _B):
                pltpu.sync_copy(b0, acc.at[pl.ds(sid * _RPS + t * _B, _B)])

        def load_idx(c, with_dstf):
            cbase = wbase + c * _CE
            pltpu.sync_copy(src_hbm.at[pl.ds(cbase, _CE)], src_c)
            if with_dstf:
                pltpu.sync_copy(dst1_hbm.at[pl.ds(cbase, _CE)], dst_f)
            pltpu.sync_copy(rel_hbm.at[pl.ds(cbase, _CE)],
                            rel_c.at[pl.ds(0, _CE)])

        def g_issue(tbl, j, p, bufs):
            buf, sem = bufs[p]
            pltpu.async_copy(tbl.at[src_c.at[pl.ds(j * _B, _B)]], buf, sem)

        def gq_issue(j, p):
            buf, sem = bb[p]
            pltpu.async_copy(q_hbm.at[dst_f.at[pl.ds(j * _B, _B)]], buf, sem)

        def d_issue(c, j, p):
            base = wbase + c * _CE + j * _B
            _, sem = bb[p]
            pltpu.async_copy(dst1_hbm.at[pl.ds(base, _B)], db[p], sem)

        def d_wait(p):
            _, sem = bb[p]
            pltpu.make_async_copy(
                dst1_hbm.at[pl.ds(0, _B)], db[p], sem).wait()

        def g_wait(tbl, p, bufs):
            buf, sem = bufs[p]
            pltpu.make_async_copy(
                tbl.at[src_c.at[pl.ds(0, _B)]], buf, sem).wait()

        def gq_wait(p):
            buf, sem = bb[p]
            pltpu.make_async_copy(
                q_hbm.at[dst_f.at[pl.ds(0, _B)]], buf, sem).wait()

        def sc_issue(p):
            pltpu.async_copy(wb[p], acc.at[db[p]], scs[p], add=True)

        def sc_wait(p):
            pltpu.make_async_copy(wb[p], acc.at[db[p]], scs[p]).wait()

        def sp_issue(c, j, p):
            base = wbase + c * _CE + j * _B
            pltpu.async_copy(sb[p], sc_hbm.at[pl.ds(base, _B)], sps[p])

        def sp_wait(p):
            pltpu.make_async_copy(
                sb[p], sc_hbm.at[pl.ds(0, _B)], sps[p]).wait()

        def sl_issue(c, j, p):
            base = wbase + c * _CE + j * _B
            _, sem = bb[p]
            pltpu.async_copy(sc_hbm.at[pl.ds(base, _B)], sb[p], sem)

        def sl_wait(p):
            _, sem = bb[p]
            pltpu.make_async_copy(
                sc_hbm.at[pl.ds(0, _B)], sb[p], sem).wait()

        zero_acc()
        pltpu.sync_copy(re_hbm, re_v)
        plsc.subcore_barrier()

        def edge1(c, j, p):
            ap, _ = ab[p]
            bp, _ = bb[p]

            def body(i, icarry):
                rid = rel_c[pl.ds(j * _B + i, 16)][0]
                ev = re_v[rid, :]
                zrow = zvec
                for h in range(_H):
                    kh = ap[i, pl.ds(h * _DK, _DK)]
                    qh = bp[i, pl.ds(h * _DK, _DK)]
                    t = (kh + ev) * qh
                    for pm in perms:
                        t = t + _xlane(t, pm)
                    zrow = jnp.where(lane == h, t, zrow)
                zrow = jnp.where(
                    lane8, jnp.exp(jnp.clip(zrow * 0.25, -10.0, 10.0)), 0.0)
                wb[p][i, pl.ds(0, 16)] = zrow
                sb[p][i, :] = zrow
                return icarry

            lax.fori_loop(0, _B, body, 0, unroll=2)

        def chunk1(c, carry):
            load_idx(c, True)
            g_issue(k_hbm, 0, 0, ab)
            gq_issue(0, 0)
            d_issue(c, 0, 0)

            def pair(t, pcarry):
                for half in range(2):
                    p = half
                    j = 2 * t + half
                    nj = j + 1

                    @pl.when(j >= 1)
                    def _():
                        sc_wait(1 - p)

                    @pl.when(j >= 2)
                    def _():
                        sp_wait(p)

                    @pl.when(nj < _CB)
                    def _():
                        g_issue(k_hbm, nj, 1 - p, ab)
                        gq_issue(nj, 1 - p)
                        d_issue(c, nj, 1 - p)

                    g_wait(k_hbm, p, ab)
                    gq_wait(p)
                    d_wait(p)

                    edge1(c, j, p)
                    sc_issue(p)
                    sp_issue(c, j, p)
                return pcarry

            lax.fori_loop(0, _CB // 2, pair, 0)
            sc_wait(1)
            sp_wait(0)
            sp_wait(1)
            return carry

        lax.fori_loop(0, _NCH, chunk1, 0)

        plsc.subcore_barrier()
        pltpu.sync_copy(acc.at[pl.ds(sid * _RPS, _RPS)],
                        zout_hbm.at[cid, pl.ds(sid * _RPS, _RPS)])
        plsc.subcore_barrier()
        zero_acc()
        plsc.subcore_barrier()

        def edge2(c, j, p):
            ap, _ = ab[p]

            def body(i, icarry):
                rid = rel_c[pl.ds(j * _B + i, 16)][0]
                ev = re_v[rid, :]
                srow = sb[p][i, :]
                for h in range(_H):
                    svec = _xlane(srow, hsplat[h])
                    vh = ap[i, pl.ds(h * _DK, _DK)]
                    wb[p][i, pl.ds(h * _DK, _DK)] = (vh + ev) * svec
                return icarry

            lax.fori_loop(0, _B, body, 0, unroll=2)

        def chunk2(c, carry):
            load_idx(c, False)
            g_issue(v_hbm, 0, 0, ab)
            sl_issue(c, 0, 0)
            d_issue(c, 0, 0)

            def pair(t, pcarry):
                for half in range(2):
                    p = half
                    j = 2 * t + half
                    nj = j + 1

                    @pl.when(j >= 1)
                    def _():
                        sc_wait(1 - p)

                    @pl.when(nj < _CB)
                    def _():
                        g_issue(v_hbm, nj, 1 - p, ab)
                        sl_issue(c, nj, 1 - p)
                        d_issue(c, nj, 1 - p)

                    g_wait(v_hbm, p, ab)
                    sl_wait(p)
                    d_wait(p)

                    edge2(c, j, p)
                    sc_issue(p)
                return pcarry

            lax.fori_loop(0, _CB // 2, pair, 0)
            sc_wait(1)
            return carry

        lax.fori_loop(0, _NCH, chunk2, 0)

        plsc.subcore_barrier()
        pltpu.sync_copy(acc.at[pl.ds(sid * _RPS, _RPS)],
                        mout_hbm.at[cid, pl.ds(sid * _RPS, _RPS)])

    return ek(q, k, v, src, dst1, rel, re)


def kernel(x, edge_index, rel_ids, rel_embed, layers):
    src = edge_index[0].astype(jnp.int32)
    dst1 = edge_index[1].astype(jnp.int32)
    rel = rel_ids.astype(jnp.int32)
    out = x
    for p in layers:
        q, k, v = _qkv(out, p["Wq"], p["bq"], p["Wk"], p["Wv"])
        zp, mp, _ = _edge_sc(q, k, v, src, dst1, rel, rel_embed)
        out = _post(out, mp[0, :_N], mp[1, :_N], zp[0, :_N], zp[1, :_N], p)
    return out


# speedup vs baseline: 1.4173x; 1.4173x over previous
import functools

import jax
import jax.numpy as jnp
from jax import lax
from jax.experimental import pallas as pl
from jax.experimental.pallas import tpu as pltpu
from jax.experimental.pallas import tpu_sc as plsc

_N = 10000
_E = 320000
_D = 128
_H = 8
_DK = 16
_FF = 512

_PREC = lax.Precision.HIGHEST

_NW = 32
_EPW = _E // _NW
_B = 40
_NBLK = _EPW // _B
_CW = _D + _DK
_NPAD = 10240
_RPS = _NPAD // 16


def _qkv_body(x_ref, wq_ref, bq_ref, wk_ref, wv_ref, q_ref, k_ref, v_ref):
    x = x_ref[...]
    q_ref[...] = jnp.dot(x, wq_ref[...], precision=_PREC) + bq_ref[...]
    k_ref[...] = jnp.dot(x, wk_ref[...], precision=_PREC)
    v_ref[...] = jnp.dot(x, wv_ref[...], precision=_PREC)


def _qkv(x, wq, bq, wk, wv):
    blk = 2000
    f = pl.pallas_call(
        _qkv_body,
        grid=(_N // blk,),
        in_specs=[
            pl.BlockSpec((blk, _D), lambda i: (i, 0)),
            pl.BlockSpec((_D, _D), lambda i: (0, 0)),
            pl.BlockSpec((1, _D), lambda i: (0, 0)),
            pl.BlockSpec((_D, _D), lambda i: (0, 0)),
            pl.BlockSpec((_D, _D), lambda i: (0, 0)),
        ],
        out_specs=[pl.BlockSpec((blk, _D), lambda i: (i, 0))] * 3,
        out_shape=[jax.ShapeDtypeStruct((_N, _D), jnp.float32)] * 3,
    )
    return f(x, wq, bq.reshape(1, _D), wk, wv)


def _ln(x, g, b):
    mu = jnp.mean(x, axis=-1, keepdims=True)
    var = jnp.mean((x - mu) ** 2, axis=-1, keepdims=True)
    return (x - mu) / jnp.sqrt(var + 1e-5) * g + b


def _post_body(x_ref, m0_ref, m1_ref, z0_ref, z1_ref,
               wo_ref, bo_ref, g1_ref, bb1_ref,
               w1_ref, bf1_ref, w2_ref, bf2_ref, g2_ref, bb2_ref, out_ref):
    wv = m0_ref[...] + m1_ref[...]
    z8 = z0_ref[:, :_H] + z1_ref[:, :_H]
    sel = (lax.broadcasted_iota(jnp.int32, (_H, _D), 1) // _DK
           == lax.broadcasted_iota(jnp.int32, (_H, _D), 0)).astype(jnp.float32)
    zf = jnp.dot(z8, sel, precision=_PREC)
    o = wv / zf
    a = x_ref[...] + jnp.dot(o, wo_ref[...], precision=_PREC) + bo_ref[...]
    a = _ln(a, g1_ref[...], bb1_ref[...])
    hid = jnp.maximum(jnp.dot(a, w1_ref[...], precision=_PREC) + bf1_ref[...], 0.0)
    f = jnp.dot(hid, w2_ref[...], precision=_PREC) + bf2_ref[...]
    out_ref[...] = _ln(a + f, g2_ref[...], bb2_ref[...])


def _post(x, m0, m1, z0, z1, p):
    blk = 2000
    f = pl.pallas_call(
        _post_body,
        grid=(_N // blk,),
        in_specs=[
            pl.BlockSpec((blk, _D), lambda i: (i, 0)),
            pl.BlockSpec((blk, _D), lambda i: (i, 0)),
            pl.BlockSpec((blk, _D), lambda i: (i, 0)),
            pl.BlockSpec((blk, _D), lambda i: (i, 0)),
            pl.BlockSpec((blk, _D), lambda i: (i, 0)),
            pl.BlockSpec((_D, _D), lambda i: (0, 0)),
            pl.BlockSpec((1, _D), lambda i: (0, 0)),
            pl.BlockSpec((1, _D), lambda i: (0, 0)),
            pl.BlockSpec((1, _D), lambda i: (0, 0)),
            pl.BlockSpec((_D, _FF), lambda i: (0, 0)),
            pl.BlockSpec((1, _FF), lambda i: (0, 0)),
            pl.BlockSpec((_FF, _D), lambda i: (0, 0)),
            pl.BlockSpec((1, _D), lambda i: (0, 0)),
            pl.BlockSpec((1, _D), lambda i: (0, 0)),
            pl.BlockSpec((1, _D), lambda i: (0, 0)),
        ],
        out_specs=pl.BlockSpec((blk, _D), lambda i: (i, 0)),
        out_shape=jax.ShapeDtypeStruct((_N, _D), jnp.float32),
    )
    return f(x, m0, m1, z0, z1, p["Wo"], p["bo"].reshape(1, _D),
             p["ln1_g"].reshape(1, _D), p["ln1_b"].reshape(1, _D),
             p["W1"], p["b1"].reshape(1, _FF), p["W2"],
             p["b2"].reshape(1, _D), p["ln2_g"].reshape(1, _D),
             p["ln2_b"].reshape(1, _D))


def _xlane(t, pm):
    return lax.gather(
        t, pm[:, None],
        lax.GatherDimensionNumbers(offset_dims=(), collapsed_slice_dims=(0,),
                                   start_index_map=(0,)),
        slice_sizes=(1,), mode=lax.GatherScatterMode.PROMISE_IN_BOUNDS)


_CB = 10
_CE = _CB * _B
_NCH = _EPW // _CE


def _edge_sc(q, k, v, src, dst1, rel, re):
    mesh = plsc.VectorSubcoreMesh(core_axis_name="c", subcore_axis_name="s")

    @functools.partial(
        pl.kernel,
        mesh=mesh,
        out_type=[
            jax.ShapeDtypeStruct((2, _NPAD, _D), jnp.float32),
            jax.ShapeDtypeStruct((2, _NPAD, _D), jnp.float32),
            jax.ShapeDtypeStruct((_E, _DK), jnp.float32),
        ],
        scratch_types=[
            pltpu.VMEM((_CE,), jnp.int32),
            pltpu.VMEM((_CE,), jnp.int32),
            pltpu.VMEM((_CE,), jnp.int32),
            pltpu.VMEM((_CE,), jnp.int32),
            pltpu.VMEM((_B,), jnp.int32),
            pltpu.VMEM((_B,), jnp.int32),
            pltpu.VMEM((_CE + 16,), jnp.int32),
            pltpu.VMEM((_CE + 16,), jnp.int32),
            pltpu.SemaphoreType.DMA,
            pltpu.SemaphoreType.DMA,
            pltpu.VMEM((_B, _D), jnp.float32),
            pltpu.VMEM((_B, _D), jnp.float32),
            pltpu.VMEM((_B, _D), jnp.float32),
            pltpu.VMEM((_B, _D), jnp.float32),
            pltpu.VMEM((_B, _DK), jnp.float32),
            pltpu.VMEM((_B, _DK), jnp.float32),
            pltpu.VMEM((100, _DK), jnp.float32),
            pltpu.VMEM_SHARED((_NPAD, _D), jnp.float32),
            pltpu.SemaphoreType.DMA,
            pltpu.SemaphoreType.DMA,
            pltpu.SemaphoreType.DMA,
            pltpu.SemaphoreType.DMA,
            pltpu.SemaphoreType.DMA,
            pltpu.SemaphoreType.DMA,
            pltpu.SemaphoreType.DMA,
            pltpu.SemaphoreType.DMA,
        ],
    )
    def ek(q_hbm, k_hbm, v_hbm, src_hbm, dst1_hbm, rel_hbm, re_hbm,
           zout_hbm, mout_hbm, sc_hbm,
           src_c0, src_c1, dst_f0, dst_f1, dst_s0, dst_s1, rel_c0, rel_c1,
           six0, six1, a0, a1, b0, b1,
           s0, s1, re_v,
           acc, sga0, sga1, sgb0, sgb1, ssc0, ssc1, ssp0, ssp1):
        cid = lax.axis_index("c")
        sid = lax.axis_index("s")
        wid = cid * 16 + sid
        wbase = wid * _EPW
        wrow0 = wid * (_EPW // _B)

        srcs = [src_c0, src_c1]
        dfs = [dst_f0, dst_f1]
        rels = [rel_c0, rel_c1]
        sixs = [six0, six1]
        ab = [(a0, sga0), (a1, sga1)]
        bb = [(b0, sgb0), (b1, sgb1)]
        wb = [b0, b1]
        sb = [s0, s1]
        db = [dst_s0, dst_s1]
        scs = [ssc0, ssc1]
        sps = [ssp0, ssp1]

        zvec = jnp.zeros((16,), jnp.float32)
        lane = lax.iota(jnp.int32, 16)
        perms = [lane ^ (1 << t) for t in range(4)]
        hsplat = [jnp.full((16,), h, jnp.int32) for h in range(_H)]
        lane8 = lane < _H

        def zero_fill(r, carry):
            for cc in range(_D // 16):
                b0[r, pl.ds(cc * 16, 16)] = zvec
            return carry

        def zero_acc():
            lax.fori_loop(0, _B, zero_fill, 0)
            for t in range(_RPS // _B):
                pltpu.sync_copy(b0, acc.at[pl.ds(sid * _RPS + t * _B, _B)])

        def idx_issue(c, u, with_dstf):
            cbase = wbase + c * _CE
            pltpu.async_copy(src_hbm.at[pl.ds(cbase, _CE)], srcs[u], sixs[u])
            if with_dstf:
                pltpu.async_copy(dst1_hbm.at[pl.ds(cbase, _CE)], dfs[u],
                                 sixs[u])
            pltpu.async_copy(rel_hbm.at[pl.ds(cbase, _CE)],
                             rels[u].at[pl.ds(0, _CE)], sixs[u])

        def idx_wait(u, with_dstf):
            pltpu.make_async_copy(
                src_hbm.at[pl.ds(0, _CE)], srcs[u], sixs[u]).wait()
            if with_dstf:
                pltpu.make_async_copy(
                    dst1_hbm.at[pl.ds(0, _CE)], dfs[u], sixs[u]).wait()
            pltpu.make_async_copy(
                rel_hbm.at[pl.ds(0, _CE)],
                rels[u].at[pl.ds(0, _CE)], sixs[u]).wait()

        def g_issue(tbl, j, p, bufs, u):
            buf, sem = bufs[p]
            pltpu.async_copy(tbl.at[srcs[u].at[pl.ds(j * _B, _B)]], buf, sem)

        def gq_issue(j, p, u):
            buf, sem = bb[p]
            pltpu.async_copy(q_hbm.at[dfs[u].at[pl.ds(j * _B, _B)]], buf, sem)

        def d_issue(c, j, p):
            base = wbase + c * _CE + j * _B
            _, sem = bb[p]
            pltpu.async_copy(dst1_hbm.at[pl.ds(base, _B)], db[p], sem)

        def d_wait(p):
            _, sem = bb[p]
            pltpu.make_async_copy(
                dst1_hbm.at[pl.ds(0, _B)], db[p], sem).wait()

        def g_wait(tbl, p, bufs):
            buf, sem = bufs[p]
            pltpu.make_async_copy(
                tbl.at[src_c0.at[pl.ds(0, _B)]], buf, sem).wait()

        def gq_wait(p):
            buf, sem = bb[p]
            pltpu.make_async_copy(
                q_hbm.at[dst_f0.at[pl.ds(0, _B)]], buf, sem).wait()

        def sc_issue(p):
            pltpu.async_copy(wb[p], acc.at[db[p]], scs[p], add=True)

        def sc_wait(p):
            pltpu.make_async_copy(wb[p], acc.at[db[p]], scs[p]).wait()

        def sp_issue(c, j, p):
            base = wbase + c * _CE + j * _B
            pltpu.async_copy(sb[p], sc_hbm.at[pl.ds(base, _B)], sps[p])

        def sp_wait(p):
            pltpu.make_async_copy(
                sb[p], sc_hbm.at[pl.ds(0, _B)], sps[p]).wait()

        def sl_issue(c, j, p):
            base = wbase + c * _CE + j * _B
            _, sem = bb[p]
            pltpu.async_copy(sc_hbm.at[pl.ds(base, _B)], sb[p], sem)

        def sl_wait(p):
            _, sem = bb[p]
            pltpu.make_async_copy(
                sc_hbm.at[pl.ds(0, _B)], sb[p], sem).wait()

        zero_acc()
        pltpu.sync_copy(re_hbm, re_v)
        plsc.subcore_barrier()

        def edge1(u, j, p):
            ap, _ = ab[p]
            bp, _ = bb[p]
            rel_c = rels[u]

            def body(i, icarry):
                rid = rel_c[pl.ds(j * _B + i, 16)][0]
                ev = re_v[rid, :]
                zrow = zvec
                for h in range(_H):
                    kh = ap[i, pl.ds(h * _DK, _DK)]
                    qh = bp[i, pl.ds(h * _DK, _DK)]
                    t = (kh + ev) * qh
                    for pm in perms:
                        t = t + _xlane(t, pm)
                    zrow = jnp.where(lane == h, t, zrow)
                zrow = jnp.where(
                    lane8, jnp.exp(jnp.clip(zrow * 0.25, -10.0, 10.0)), 0.0)
                wb[p][i, pl.ds(0, 16)] = zrow
                sb[p][i, :] = zrow
                return icarry

            lax.fori_loop(0, _B, body, 0)

        def chunk1(c, u):
            idx_wait(u, True)
            g_issue(k_hbm, 0, 0, ab, u)
            gq_issue(0, 0, u)
            d_issue(c, 0, 0)

            @pl.when(c + 1 < _NCH)
            def _():
                idx_issue(c + 1, 1 - u, True)

            def pair(t, pcarry):
                for half in range(2):
                    p = half
                    j = 2 * t + half
                    nj = j + 1

                    @pl.when(j >= 1)
                    def _():
                        sc_wait(1 - p)

                    @pl.when(j >= 2)
                    def _():
                        sp_wait(p)

                    @pl.when(nj < _CB)
                    def _():
                        g_issue(k_hbm, nj, 1 - p, ab, u)
                        gq_issue(nj, 1 - p, u)
                        d_issue(c, nj, 1 - p)

                    g_wait(k_hbm, p, ab)
                    gq_wait(p)
                    d_wait(p)

                    edge1(u, j, p)
                    sc_issue(p)
                    sp_issue(c, j, p)
                return pcarry

            lax.fori_loop(0, _CB // 2, pair, 0)
            sc_wait(1)
            sp_wait(0)
            sp_wait(1)

        idx_issue(0, 0, True)

        def cpair1(t, carry):
            chunk1(2 * t, 0)
            chunk1(2 * t + 1, 1)
            return carry

        lax.fori_loop(0, _NCH // 2, cpair1, 0)
        chunk1(_NCH - 1, 0)

        plsc.subcore_barrier()
        pltpu.sync_copy(acc.at[pl.ds(sid * _RPS, _RPS)],
                        zout_hbm.at[cid, pl.ds(sid * _RPS, _RPS)])
        plsc.subcore_barrier()
        zero_acc()
        plsc.subcore_barrier()

        def edge2(u, j, p):
            ap, _ = ab[p]
            rel_c = rels[u]

            def body(i, icarry):
                rid = rel_c[pl.ds(j * _B + i, 16)][0]
                ev = re_v[rid, :]
                srow = sb[p][i, :]
                for h in range(_H):
                    svec = _xlane(srow, hsplat[h])
                    vh = ap[i, pl.ds(h * _DK, _DK)]
                    wb[p][i, pl.ds(h * _DK, _DK)] = (vh + ev) * svec
                return icarry

            lax.fori_loop(0, _B, body, 0)

        def chunk2(c, u):
            idx_wait(u, False)
            g_issue(v_hbm, 0, 0, ab, u)
            sl_issue(c, 0, 0)
            d_issue(c, 0, 0)

            @pl.when(c + 1 < _NCH)
            def _():
                idx_issue(c + 1, 1 - u, False)

            def pair(t, pcarry):
                for half in range(2):
                    p = half
                    j = 2 * t + half
                    nj = j + 1

                    @pl.when(j >= 1)
                    def _():
                        sc_wait(1 - p)

                    @pl.when(nj < _CB)
                    def _():
                        g_issue(v_hbm, nj, 1 - p, ab, u)
                        sl_issue(c, nj, 1 - p)
                        d_issue(c, nj, 1 - p)

                    g_wait(v_hbm, p, ab)
                    sl_wait(p)
                    d_wait(p)

                    edge2(u, j, p)
                    sc_issue(p)
                return pcarry

            lax.fori_loop(0, _CB // 2, pair, 0)
            sc_wait(1)

        idx_issue(0, 0, False)

        def cpair2(t, carry):
            chunk2(2 * t, 0)
            chunk2(2 * t + 1, 1)
            return carry

        lax.fori_loop(0, _NCH // 2, cpair2, 0)
        chunk2(_NCH - 1, 0)

        plsc.subcore_barrier()
        pltpu.sync_copy(acc.at[pl.ds(sid * _RPS, _RPS)],
                        mout_hbm.at[cid, pl.ds(sid * _RPS, _RPS)])

    return ek(q, k, v, src, dst1, rel, re)


def kernel(x, edge_index, rel_ids, rel_embed, layers):
    src = edge_index[0].astype(jnp.int32)
    dst1 = edge_index[1].astype(jnp.int32)
    rel = rel_ids.astype(jnp.int32)
    out = x
    for p in layers:
        q, k, v = _qkv(out, p["Wq"], p["bq"], p["Wk"], p["Wv"])
        zp, mp, _ = _edge_sc(q, k, v, src, dst1, rel, rel_embed)
        out = _post(out, mp[0, :_N], mp[1, :_N], zp[0, :_N], zp[1, :_N], p)
    return out


# speedup vs baseline: 1.4193x; 1.0014x over previous
import functools

import jax
import jax.numpy as jnp
from jax import lax
from jax.experimental import pallas as pl
from jax.experimental.pallas import tpu as pltpu
from jax.experimental.pallas import tpu_sc as plsc

_N = 10000
_E = 320000
_D = 128
_H = 8
_DK = 16
_FF = 512

_PREC = lax.Precision.HIGHEST

_NW = 32
_EPW = _E // _NW
_B = 40
_NBLK = _EPW // _B
_CW = _D + _DK
_NPAD = 10240
_RPS = _NPAD // 16


def _qkv_body(x_ref, wq_ref, bq_ref, wk_ref, wv_ref, q_ref, k_ref, v_ref):
    x = x_ref[...]
    q_ref[...] = jnp.dot(x, wq_ref[...], precision=_PREC) + bq_ref[...]
    k_ref[...] = jnp.dot(x, wk_ref[...], precision=_PREC)
    v_ref[...] = jnp.dot(x, wv_ref[...], precision=_PREC)


def _qkv(x, wq, bq, wk, wv):
    blk = 2000
    f = pl.pallas_call(
        _qkv_body,
        grid=(_N // blk,),
        in_specs=[
            pl.BlockSpec((blk, _D), lambda i: (i, 0)),
            pl.BlockSpec((_D, _D), lambda i: (0, 0)),
            pl.BlockSpec((1, _D), lambda i: (0, 0)),
            pl.BlockSpec((_D, _D), lambda i: (0, 0)),
            pl.BlockSpec((_D, _D), lambda i: (0, 0)),
        ],
        out_specs=[pl.BlockSpec((blk, _D), lambda i: (i, 0))] * 3,
        out_shape=[jax.ShapeDtypeStruct((_N, _D), jnp.float32)] * 3,
    )
    return f(x, wq, bq.reshape(1, _D), wk, wv)


def _ln(x, g, b):
    mu = jnp.mean(x, axis=-1, keepdims=True)
    var = jnp.mean((x - mu) ** 2, axis=-1, keepdims=True)
    return (x - mu) / jnp.sqrt(var + 1e-5) * g + b


def _post_body(x_ref, m0_ref, m1_ref, z0_ref, z1_ref,
               wo_ref, bo_ref, g1_ref, bb1_ref,
               w1_ref, bf1_ref, w2_ref, bf2_ref, g2_ref, bb2_ref, out_ref):
    wv = m0_ref[...] + m1_ref[...]
    z8 = z0_ref[:, :_H] + z1_ref[:, :_H]
    sel = (lax.broadcasted_iota(jnp.int32, (_H, _D), 1) // _DK
           == lax.broadcasted_iota(jnp.int32, (_H, _D), 0)).astype(jnp.float32)
    zf = jnp.dot(z8, sel, precision=_PREC)
    o = wv / zf
    a = x_ref[...] + jnp.dot(o, wo_ref[...], precision=_PREC) + bo_ref[...]
    a = _ln(a, g1_ref[...], bb1_ref[...])
    hid = jnp.maximum(jnp.dot(a, w1_ref[...], precision=_PREC) + bf1_ref[...], 0.0)
    f = jnp.dot(hid, w2_ref[...], precision=_PREC) + bf2_ref[...]
    out_ref[...] = _ln(a + f, g2_ref[...], bb2_ref[...])


def _post(x, m0, m1, z0, z1, p):
    blk = 2000
    f = pl.pallas_call(
        _post_body,
        grid=(_N // blk,),
        in_specs=[
            pl.BlockSpec((blk, _D), lambda i: (i, 0)),
            pl.BlockSpec((blk, _D), lambda i: (i, 0)),
            pl.BlockSpec((blk, _D), lambda i: (i, 0)),
            pl.BlockSpec((blk, _D), lambda i: (i, 0)),
            pl.BlockSpec((blk, _D), lambda i: (i, 0)),
            pl.BlockSpec((_D, _D), lambda i: (0, 0)),
            pl.BlockSpec((1, _D), lambda i: (0, 0)),
            pl.BlockSpec((1, _D), lambda i: (0, 0)),
            pl.BlockSpec((1, _D), lambda i: (0, 0)),
            pl.BlockSpec((_D, _FF), lambda i: (0, 0)),
            pl.BlockSpec((1, _FF), lambda i: (0, 0)),
            pl.BlockSpec((_FF, _D), lambda i: (0, 0)),
            pl.BlockSpec((1, _D), lambda i: (0, 0)),
            pl.BlockSpec((1, _D), lambda i: (0, 0)),
            pl.BlockSpec((1, _D), lambda i: (0, 0)),
        ],
        out_specs=pl.BlockSpec((blk, _D), lambda i: (i, 0)),
        out_shape=jax.ShapeDtypeStruct((_N, _D), jnp.float32),
    )
    return f(x, m0, m1, z0, z1, p["Wo"], p["bo"].reshape(1, _D),
             p["ln1_g"].reshape(1, _D), p["ln1_b"].reshape(1, _D),
             p["W1"], p["b1"].reshape(1, _FF), p["W2"],
             p["b2"].reshape(1, _D), p["ln2_g"].reshape(1, _D),
             p["ln2_b"].reshape(1, _D))


def _xlane(t, pm):
    return lax.gather(
        t, pm[:, None],
        lax.GatherDimensionNumbers(offset_dims=(), collapsed_slice_dims=(0,),
                                   start_index_map=(0,)),
        slice_sizes=(1,), mode=lax.GatherScatterMode.PROMISE_IN_BOUNDS)


_CB = 10
_CE = _CB * _B
_NCH = _EPW // _CE


def _edge_sc(q, k, v, src, dst1, rel, re):
    mesh = plsc.VectorSubcoreMesh(core_axis_name="c", subcore_axis_name="s")

    @functools.partial(
        pl.kernel,
        mesh=mesh,
        out_type=[
            jax.ShapeDtypeStruct((2, _NPAD, _D), jnp.float32),
            jax.ShapeDtypeStruct((2, _NPAD, _D), jnp.float32),
            jax.ShapeDtypeStruct((_E, _DK), jnp.float32),
        ],
        scratch_types=[
            pltpu.VMEM((_CE,), jnp.int32),
            pltpu.VMEM((_CE,), jnp.int32),
            pltpu.VMEM((_CE,), jnp.int32),
            pltpu.VMEM((_CE,), jnp.int32),
            pltpu.VMEM((_B,), jnp.int32),
            pltpu.VMEM((_B,), jnp.int32),
            pltpu.VMEM((_CE + 16,), jnp.int32),
            pltpu.VMEM((_CE + 16,), jnp.int32),
            pltpu.SemaphoreType.DMA,
            pltpu.SemaphoreType.DMA,
            pltpu.VMEM((_B, _D), jnp.float32),
            pltpu.VMEM((_B, _D), jnp.float32),
            pltpu.VMEM((_B, _D), jnp.float32),
            pltpu.VMEM((_B, _D), jnp.float32),
            pltpu.VMEM((_B, _DK), jnp.float32),
            pltpu.VMEM((_B, _DK), jnp.float32),
            pltpu.VMEM((100, _DK), jnp.float32),
            pltpu.VMEM_SHARED((_NPAD, _D), jnp.float32),
            pltpu.SemaphoreType.DMA,
            pltpu.SemaphoreType.DMA,
            pltpu.SemaphoreType.DMA,
            pltpu.SemaphoreType.DMA,
            pltpu.SemaphoreType.DMA,
            pltpu.SemaphoreType.DMA,
            pltpu.SemaphoreType.DMA,
            pltpu.SemaphoreType.DMA,
        ],
    )
    def ek(q_hbm, k_hbm, v_hbm, src_hbm, dst1_hbm, rel_hbm, re_hbm,
           zout_hbm, mout_hbm, sc_hbm,
           src_c0, src_c1, dst_f0, dst_f1, dst_s0, dst_s1, rel_c0, rel_c1,
           six0, six1, a0, a1, b0, b1,
           s0, s1, re_v,
           acc, sga0, sga1, sgb0, sgb1, ssc0, ssc1, ssp0, ssp1):
        cid = lax.axis_index("c")
        sid = lax.axis_index("s")
        wid = cid * 16 + sid
        wbase = wid * _EPW
        wrow0 = wid * (_EPW // _B)

        srcs = [src_c0, src_c1]
        dfs = [dst_f0, dst_f1]
        rels = [rel_c0, rel_c1]
        sixs = [six0, six1]
        ab = [(a0, sga0), (a1, sga1)]
        bb = [(b0, sgb0), (b1, sgb1)]
        wb = [b0, b1]
        sb = [s0, s1]
        db = [dst_s0, dst_s1]
        scs = [ssc0, ssc1]
        sps = [ssp0, ssp1]

        zvec = jnp.zeros((16,), jnp.float32)
        lane = lax.iota(jnp.int32, 16)
        perms = [lane ^ (1 << t) for t in range(4)]
        hsplat = [jnp.full((16,), h, jnp.int32) for h in range(_H)]
        lane8 = lane < _H

        def zero_fill(r, carry):
            for cc in range(_D // 16):
                b0[r, pl.ds(cc * 16, 16)] = zvec
            return carry

        def zero_acc():
            lax.fori_loop(0, _B, zero_fill, 0)
            for t in range(_RPS // _B):
                pltpu.async_copy(
                    b0, acc.at[pl.ds(sid * _RPS + t * _B, _B)], sga0)
            for t in range(_RPS // _B):
                pltpu.make_async_copy(
                    b0, acc.at[pl.ds(sid * _RPS, _B)], sga0).wait()

        def idx_issue(c, u, with_dstf):
            cbase = wbase + c * _CE
            pltpu.async_copy(src_hbm.at[pl.ds(cbase, _CE)], srcs[u], sixs[u])
            if with_dstf:
                pltpu.async_copy(dst1_hbm.at[pl.ds(cbase, _CE)], dfs[u],
                                 sixs[u])
            pltpu.async_copy(rel_hbm.at[pl.ds(cbase, _CE)],
                             rels[u].at[pl.ds(0, _CE)], sixs[u])

        def idx_wait(u, with_dstf):
            pltpu.make_async_copy(
                src_hbm.at[pl.ds(0, _CE)], srcs[u], sixs[u]).wait()
            if with_dstf:
                pltpu.make_async_copy(
                    dst1_hbm.at[pl.ds(0, _CE)], dfs[u], sixs[u]).wait()
            pltpu.make_async_copy(
                rel_hbm.at[pl.ds(0, _CE)],
                rels[u].at[pl.ds(0, _CE)], sixs[u]).wait()

        def g_issue(tbl, j, p, bufs, u):
            buf, sem = bufs[p]
            pltpu.async_copy(tbl.at[srcs[u].at[pl.ds(j * _B, _B)]], buf, sem)

        def gq_issue(j, p, u):
            buf, sem = bb[p]
            pltpu.async_copy(q_hbm.at[dfs[u].at[pl.ds(j * _B, _B)]], buf, sem)

        def d_issue(c, j, p):
            base = wbase + c * _CE + j * _B
            _, sem = bb[p]
            pltpu.async_copy(dst1_hbm.at[pl.ds(base, _B)], db[p], sem)

        def d_wait(p):
            _, sem = bb[p]
            pltpu.make_async_copy(
                dst1_hbm.at[pl.ds(0, _B)], db[p], sem).wait()

        def g_wait(tbl, p, bufs):
            buf, sem = bufs[p]
            pltpu.make_async_copy(
                tbl.at[src_c0.at[pl.ds(0, _B)]], buf, sem).wait()

        def gq_wait(p):
            buf, sem = bb[p]
            pltpu.make_async_copy(
                q_hbm.at[dst_f0.at[pl.ds(0, _B)]], buf, sem).wait()

        def sc_issue(p):
            pltpu.async_copy(wb[p], acc.at[db[p]], scs[p], add=True)

        def sc_wait(p):
            pltpu.make_async_copy(wb[p], acc.at[db[p]], scs[p]).wait()

        def sp_issue(c, j, p):
            base = wbase + c * _CE + j * _B
            pltpu.async_copy(sb[p], sc_hbm.at[pl.ds(base, _B)], sps[p])

        def sp_wait(p):
            pltpu.make_async_copy(
                sb[p], sc_hbm.at[pl.ds(0, _B)], sps[p]).wait()

        def sl_issue(c, j, p):
            base = wbase + c * _CE + j * _B
            _, sem = bb[p]
            pltpu.async_copy(sc_hbm.at[pl.ds(base, _B)], sb[p], sem)

        def sl_wait(p):
            _, sem = bb[p]
            pltpu.make_async_copy(
                sc_hbm.at[pl.ds(0, _B)], sb[p], sem).wait()

        zero_acc()
        pltpu.sync_copy(re_hbm, re_v)
        plsc.subcore_barrier()

        def edge1(u, j, p):
            ap, _ = ab[p]
            bp, _ = bb[p]
            rel_c = rels[u]

            def body(i, icarry):
                rid = rel_c[pl.ds(j * _B + i, 16)][0]
                ev = re_v[rid, :]
                zrow = zvec
                for h in range(_H):
                    kh = ap[i, pl.ds(h * _DK, _DK)]
                    qh = bp[i, pl.ds(h * _DK, _DK)]
                    t = (kh + ev) * qh
                    for pm in perms:
                        t = t + _xlane(t, pm)
                    zrow = jnp.where(lane == h, t, zrow)
                zrow = jnp.where(
                    lane8, jnp.exp(jnp.clip(zrow * 0.25, -10.0, 10.0)), 0.0)
                wb[p][i, pl.ds(0, 16)] = zrow
                sb[p][i, :] = zrow
                return icarry

            lax.fori_loop(0, _B, body, 0)

        def chunk1(c, u):
            idx_wait(u, True)
            g_issue(k_hbm, 0, 0, ab, u)
            gq_issue(0, 0, u)
            d_issue(c, 0, 0)

            @pl.when(c + 1 < _NCH)
            def _():
                idx_issue(c + 1, 1 - u, True)

            def pair(t, pcarry):
                for half in range(2):
                    p = half
                    j = 2 * t + half
                    nj = j + 1

                    @pl.when(j >= 1)
                    def _():
                        sc_wait(1 - p)

                    @pl.when(j >= 2)
                    def _():
                        sp_wait(p)

                    @pl.when(nj < _CB)
                    def _():
                        g_issue(k_hbm, nj, 1 - p, ab, u)
                        gq_issue(nj, 1 - p, u)
                        d_issue(c, nj, 1 - p)

                    g_wait(k_hbm, p, ab)
                    gq_wait(p)
                    d_wait(p)

                    edge1(u, j, p)
                    sc_issue(p)
                    sp_issue(c, j, p)
                return pcarry

            lax.fori_loop(0, _CB // 2, pair, 0)
            sc_wait(1)
            sp_wait(0)
            sp_wait(1)

        idx_issue(0, 0, True)

        def cpair1(t, carry):
            chunk1(2 * t, 0)
            chunk1(2 * t + 1, 1)
            return carry

        lax.fori_loop(0, _NCH // 2, cpair1, 0)
        chunk1(_NCH - 1, 0)

        plsc.subcore_barrier()
        pltpu.sync_copy(acc.at[pl.ds(sid * _RPS, _RPS)],
                        zout_hbm.at[cid, pl.ds(sid * _RPS, _RPS)])
        plsc.subcore_barrier()
        zero_acc()
        plsc.subcore_barrier()

        def edge2(u, j, p):
            ap, _ = ab[p]
            rel_c = rels[u]

            def body(i, icarry):
                rid = rel_c[pl.ds(j * _B + i, 16)][0]
                ev = re_v[rid, :]
                srow = sb[p][i, :]
                for h in range(_H):
                    svec = _xlane(srow, hsplat[h])
                    vh = ap[i, pl.ds(h * _DK, _DK)]
                    wb[p][i, pl.ds(h * _DK, _DK)] = (vh + ev) * svec
                return icarry

            lax.fori_loop(0, _B, body, 0)

        def chunk2(c, u):
            idx_wait(u, False)
            g_issue(v_hbm, 0, 0, ab, u)
            sl_issue(c, 0, 0)
            d_issue(c, 0, 0)

            @pl.when(c + 1 < _NCH)
            def _():
                idx_issue(c + 1, 1 - u, False)

            def pair(t, pcarry):
                for half in range(2):
                    p = half
                    j = 2 * t + half
                    nj = j + 1

                    @pl.when(j >= 1)
                    def _():
                        sc_wait(1 - p)

                    @pl.when(nj < _CB)
                    def _():
                        g_issue(v_hbm, nj, 1 - p, ab, u)
                        sl_issue(c, nj, 1 - p)
                        d_issue(c, nj, 1 - p)

                    g_wait(v_hbm, p, ab)
                    sl_wait(p)
                    d_wait(p)

                    edge2(u, j, p)
                    sc_issue(p)
                return pcarry

            lax.fori_loop(0, _CB // 2, pair, 0)
            sc_wait(1)

        idx_issue(0, 0, False)

        def cpair2(t, carry):
            chunk2(2 * t, 0)
            chunk2(2 * t + 1, 1)
            return carry

        lax.fori_loop(0, _NCH // 2, cpair2, 0)
        chunk2(_NCH - 1, 0)

        plsc.subcore_barrier()
        pltpu.sync_copy(acc.at[pl.ds(sid * _RPS, _RPS)],
                        mout_hbm.at[cid, pl.ds(sid * _RPS, _RPS)])

    return ek(q, k, v, src, dst1, rel, re)


def kernel(x, edge_index, rel_ids, rel_embed, layers):
    src = edge_index[0].astype(jnp.int32)
    dst1 = edge_index[1].astype(jnp.int32)
    rel = rel_ids.astype(jnp.int32)
    out = x
    for p in layers:
        q, k, v = _qkv(out, p["Wq"], p["bq"], p["Wk"], p["Wv"])
        zp, mp, _ = _edge_sc(q, k, v, src, dst1, rel, rel_embed)
        out = _post(out, mp[0, :_N], mp[1, :_N], zp[0, :_N], zp[1, :_N], p)
    return out


# speedup vs baseline: 2.1822x; 1.5375x over previous
import functools

import jax
import jax.numpy as jnp
from jax import lax
from jax.experimental import pallas as pl
from jax.experimental.pallas import tpu as pltpu
from jax.experimental.pallas import tpu_sc as plsc

_N = 10000
_E = 320000
_D = 128
_H = 8
_DK = 16
_FF = 512

_PREC = lax.Precision.HIGHEST

_NW = 32
_EPW = _E // _NW
_B = 40
_NBLK = _EPW // _B
_CW = _D + _DK
_NPAD = 10240
_RPS = _NPAD // 16


def _qkv_body(x_ref, wq_ref, bq_ref, wk_ref, wv_ref, q_ref, k_ref, v_ref):
    x = x_ref[...]
    q_ref[...] = jnp.dot(x, wq_ref[...], precision=_PREC) + bq_ref[...]
    k_ref[...] = jnp.dot(x, wk_ref[...], precision=_PREC)
    v_ref[...] = jnp.dot(x, wv_ref[...], precision=_PREC)


def _qkv(x, wq, bq, wk, wv):
    blk = 2000
    f = pl.pallas_call(
        _qkv_body,
        grid=(_N // blk,),
        in_specs=[
            pl.BlockSpec((blk, _D), lambda i: (i, 0)),
            pl.BlockSpec((_D, _D), lambda i: (0, 0)),
            pl.BlockSpec((1, _D), lambda i: (0, 0)),
            pl.BlockSpec((_D, _D), lambda i: (0, 0)),
            pl.BlockSpec((_D, _D), lambda i: (0, 0)),
        ],
        out_specs=[pl.BlockSpec((blk, _D), lambda i: (i, 0))] * 3,
        out_shape=[jax.ShapeDtypeStruct((_N, _D), jnp.float32)] * 3,
    )
    return f(x, wq, bq.reshape(1, _D), wk, wv)


def _ln(x, g, b):
    mu = jnp.mean(x, axis=-1, keepdims=True)
    var = jnp.mean((x - mu) ** 2, axis=-1, keepdims=True)
    return (x - mu) / jnp.sqrt(var + 1e-5) * g + b


def _post_body(x_ref, m0_ref, m1_ref, z0_ref, z1_ref,
               wo_ref, bo_ref, g1_ref, bb1_ref,
               w1_ref, bf1_ref, w2_ref, bf2_ref, g2_ref, bb2_ref, out_ref):
    wv = m0_ref[...] + m1_ref[...]
    z8 = z0_ref[:, :_H] + z1_ref[:, :_H]
    sel = (lax.broadcasted_iota(jnp.int32, (_H, _D), 1) // _DK
           == lax.broadcasted_iota(jnp.int32, (_H, _D), 0)).astype(jnp.float32)
    zf = jnp.dot(z8, sel, precision=_PREC)
    o = wv / zf
    a = x_ref[...] + jnp.dot(o, wo_ref[...], precision=_PREC) + bo_ref[...]
    a = _ln(a, g1_ref[...], bb1_ref[...])
    hid = jnp.maximum(jnp.dot(a, w1_ref[...], precision=_PREC) + bf1_ref[...], 0.0)
    f = jnp.dot(hid, w2_ref[...], precision=_PREC) + bf2_ref[...]
    out_ref[...] = _ln(a + f, g2_ref[...], bb2_ref[...])


def _post(x, m0, m1, z0, z1, p):
    blk = 2000
    f = pl.pallas_call(
        _post_body,
        grid=(_N // blk,),
        in_specs=[
            pl.BlockSpec((blk, _D), lambda i: (i, 0)),
            pl.BlockSpec((blk, _D), lambda i: (i, 0)),
            pl.BlockSpec((blk, _D), lambda i: (i, 0)),
            pl.BlockSpec((blk, _D), lambda i: (i, 0)),
            pl.BlockSpec((blk, _D), lambda i: (i, 0)),
            pl.BlockSpec((_D, _D), lambda i: (0, 0)),
            pl.BlockSpec((1, _D), lambda i: (0, 0)),
            pl.BlockSpec((1, _D), lambda i: (0, 0)),
            pl.BlockSpec((1, _D), lambda i: (0, 0)),
            pl.BlockSpec((_D, _FF), lambda i: (0, 0)),
            pl.BlockSpec((1, _FF), lambda i: (0, 0)),
            pl.BlockSpec((_FF, _D), lambda i: (0, 0)),
            pl.BlockSpec((1, _D), lambda i: (0, 0)),
            pl.BlockSpec((1, _D), lambda i: (0, 0)),
            pl.BlockSpec((1, _D), lambda i: (0, 0)),
        ],
        out_specs=pl.BlockSpec((blk, _D), lambda i: (i, 0)),
        out_shape=jax.ShapeDtypeStruct((_N, _D), jnp.float32),
    )
    return f(x, m0, m1, z0, z1, p["Wo"], p["bo"].reshape(1, _D),
             p["ln1_g"].reshape(1, _D), p["ln1_b"].reshape(1, _D),
             p["W1"], p["b1"].reshape(1, _FF), p["W2"],
             p["b2"].reshape(1, _D), p["ln2_g"].reshape(1, _D),
             p["ln2_b"].reshape(1, _D))


def _xlane(t, pm):
    return lax.gather(
        t, pm[:, None],
        lax.GatherDimensionNumbers(offset_dims=(), collapsed_slice_dims=(0,),
                                   start_index_map=(0,)),
        slice_sizes=(1,), mode=lax.GatherScatterMode.PROMISE_IN_BOUNDS)


_CB = 10
_CE = _CB * _B
_NCH = _EPW // _CE


def _edge_sc(q, k, v, src, dst1, rel, re):
    mesh = plsc.VectorSubcoreMesh(core_axis_name="c", subcore_axis_name="s")

    @functools.partial(
        pl.kernel,
        mesh=mesh,
        out_type=[
            jax.ShapeDtypeStruct((2, _NPAD, _D), jnp.float32),
            jax.ShapeDtypeStruct((2, _NPAD, _D), jnp.float32),
            jax.ShapeDtypeStruct((_E, _DK), jnp.float32),
        ],
        scratch_types=[
            pltpu.VMEM((_CE,), jnp.int32),
            pltpu.VMEM((_CE,), jnp.int32),
            pltpu.VMEM((_CE,), jnp.int32),
            pltpu.VMEM((_CE,), jnp.int32),
            pltpu.VMEM((_B,), jnp.int32),
            pltpu.VMEM((_B,), jnp.int32),
            pltpu.VMEM((_CE + 16,), jnp.int32),
            pltpu.VMEM((_CE + 16,), jnp.int32),
            pltpu.SemaphoreType.DMA,
            pltpu.SemaphoreType.DMA,
            pltpu.VMEM((_B, _D), jnp.float32),
            pltpu.VMEM((_B, _D), jnp.float32),
            pltpu.VMEM((_B, _D), jnp.float32),
            pltpu.VMEM((_B, _D), jnp.float32),
            pltpu.VMEM((_B, _DK), jnp.float32),
            pltpu.VMEM((_B, _DK), jnp.float32),
            pltpu.VMEM((100, _DK), jnp.float32),
            pltpu.VMEM_SHARED((_NPAD, _D), jnp.float32),
            pltpu.SemaphoreType.DMA,
            pltpu.SemaphoreType.DMA,
            pltpu.SemaphoreType.DMA,
            pltpu.SemaphoreType.DMA,
            pltpu.SemaphoreType.DMA,
            pltpu.SemaphoreType.DMA,
            pltpu.SemaphoreType.DMA,
            pltpu.SemaphoreType.DMA,
        ],
    )
    def ek(q_hbm, k_hbm, v_hbm, src_hbm, dst1_hbm, rel_hbm, re_hbm,
           zout_hbm, mout_hbm, sc_hbm,
           src_c0, src_c1, dst_f0, dst_f1, dst_s0, dst_s1, rel_c0, rel_c1,
           six0, six1, a0, a1, b0, b1,
           s0, s1, re_v,
           acc, sga0, sga1, sgb0, sgb1, ssc0, ssc1, ssp0, ssp1):
        cid = lax.axis_index("c")
        sid = lax.axis_index("s")
        wid = cid * 16 + sid
        wbase = wid * _EPW
        wrow0 = wid * (_EPW // _B)

        srcs = [src_c0, src_c1]
        dfs = [dst_f0, dst_f1]
        rels = [rel_c0, rel_c1]
        sixs = [six0, six1]
        ab = [(a0, sga0), (a1, sga1)]
        bb = [(b0, sgb0), (b1, sgb1)]
        wb = [b0, b1]
        sb = [s0, s1]
        db = [dst_s0, dst_s1]
        scs = [ssc0, ssc1]
        sps = [ssp0, ssp1]

        zvec = jnp.zeros((16,), jnp.float32)
        lane = lax.iota(jnp.int32, 16)
        perms = [lane ^ (1 << t) for t in range(4)]
        hsplat = [jnp.full((16,), h, jnp.int32) for h in range(_H)]
        lane8 = lane < _H

        def zero_fill(r, carry):
            for cc in range(_D // 16):
                b0[r, pl.ds(cc * 16, 16)] = zvec
            return carry

        def zero_acc():
            lax.fori_loop(0, _B, zero_fill, 0)
            for t in range(_RPS // _B):
                pltpu.async_copy(
                    b0, acc.at[pl.ds(sid * _RPS + t * _B, _B)], sga0)
            for t in range(_RPS // _B):
                pltpu.make_async_copy(
                    b0, acc.at[pl.ds(sid * _RPS, _B)], sga0).wait()

        def idx_issue(c, u, with_dstf):
            cbase = wbase + c * _CE
            pltpu.async_copy(src_hbm.at[pl.ds(cbase, _CE)], srcs[u], sixs[u])
            if with_dstf:
                pltpu.async_copy(dst1_hbm.at[pl.ds(cbase, _CE)], dfs[u],
                                 sixs[u])
            pltpu.async_copy(rel_hbm.at[pl.ds(cbase, _CE)],
                             rels[u].at[pl.ds(0, _CE)], sixs[u])

        def idx_wait(u, with_dstf):
            pltpu.make_async_copy(
                src_hbm.at[pl.ds(0, _CE)], srcs[u], sixs[u]).wait()
            if with_dstf:
                pltpu.make_async_copy(
                    dst1_hbm.at[pl.ds(0, _CE)], dfs[u], sixs[u]).wait()
            pltpu.make_async_copy(
                rel_hbm.at[pl.ds(0, _CE)],
                rels[u].at[pl.ds(0, _CE)], sixs[u]).wait()

        def g_issue(tbl, j, p, bufs, u):
            buf, sem = bufs[p]
            pltpu.async_copy(tbl.at[srcs[u].at[pl.ds(j * _B, _B)]], buf, sem)

        def gq_issue(j, p, u):
            buf, sem = bb[p]
            pltpu.async_copy(q_hbm.at[dfs[u].at[pl.ds(j * _B, _B)]], buf, sem)

        def d_issue(c, j, p):
            base = wbase + c * _CE + j * _B
            _, sem = bb[p]
            pltpu.async_copy(dst1_hbm.at[pl.ds(base, _B)], db[p], sem)

        def d_wait(p):
            _, sem = bb[p]
            pltpu.make_async_copy(
                dst1_hbm.at[pl.ds(0, _B)], db[p], sem).wait()

        def g_wait(tbl, p, bufs):
            buf, sem = bufs[p]
            pltpu.make_async_copy(
                tbl.at[src_c0.at[pl.ds(0, _B)]], buf, sem).wait()

        def gq_wait(p):
            buf, sem = bb[p]
            pltpu.make_async_copy(
                q_hbm.at[dst_f0.at[pl.ds(0, _B)]], buf, sem).wait()

        def sc_issue(p):
            pltpu.async_copy(wb[p], acc.at[db[p]], scs[p], add=True)

        def sc_wait(p):
            pltpu.make_async_copy(wb[p], acc.at[db[p]], scs[p]).wait()

        def sp_issue(c, j, p):
            base = wbase + c * _CE + j * _B
            pltpu.async_copy(sb[p], sc_hbm.at[pl.ds(base, _B)], sps[p])

        def sp_wait(p):
            pltpu.make_async_copy(
                sb[p], sc_hbm.at[pl.ds(0, _B)], sps[p]).wait()

        def sl_issue(c, j, p):
            base = wbase + c * _CE + j * _B
            _, sem = bb[p]
            pltpu.async_copy(sc_hbm.at[pl.ds(base, _B)], sb[p], sem)

        def sl_wait(p):
            _, sem = bb[p]
            pltpu.make_async_copy(
                sc_hbm.at[pl.ds(0, _B)], sb[p], sem).wait()

        zero_acc()
        pltpu.sync_copy(re_hbm, re_v)
        plsc.subcore_barrier()

        def edge1(u, j, p):
            ap, _ = ab[p]
            bp, _ = bb[p]
            rel_c = rels[u]

            @plsc.parallel_loop(0, _B, unroll=4)
            def body(i):
                rid = rel_c[pl.ds(j * _B + i, 16)][0]
                ev = re_v[rid, :]
                zrow = zvec
                for h in range(_H):
                    kh = ap[i, pl.ds(h * _DK, _DK)]
                    qh = bp[i, pl.ds(h * _DK, _DK)]
                    t = (kh + ev) * qh
                    for pm in perms:
                        t = t + _xlane(t, pm)
                    zrow = jnp.where(lane == h, t, zrow)
                zrow = jnp.where(
                    lane8, jnp.exp(jnp.clip(zrow * 0.25, -10.0, 10.0)), 0.0)
                wb[p][i, pl.ds(0, 16)] = zrow
                sb[p][i, :] = zrow

        def chunk1(c, u):
            idx_wait(u, True)
            g_issue(k_hbm, 0, 0, ab, u)
            gq_issue(0, 0, u)
            d_issue(c, 0, 0)

            @pl.when(c + 1 < _NCH)
            def _():
                idx_issue(c + 1, 1 - u, True)

            def pair(t, pcarry):
                for half in range(2):
                    p = half
                    j = 2 * t + half
                    nj = j + 1

                    @pl.when(j >= 1)
                    def _():
                        sc_wait(1 - p)

                    @pl.when(j >= 2)
                    def _():
                        sp_wait(p)

                    @pl.when(nj < _CB)
                    def _():
                        g_issue(k_hbm, nj, 1 - p, ab, u)
                        gq_issue(nj, 1 - p, u)
                        d_issue(c, nj, 1 - p)

                    g_wait(k_hbm, p, ab)
                    gq_wait(p)
                    d_wait(p)

                    edge1(u, j, p)
                    sc_issue(p)
                    sp_issue(c, j, p)
                return pcarry

            lax.fori_loop(0, _CB // 2, pair, 0)
            sc_wait(1)
            sp_wait(0)
            sp_wait(1)

        idx_issue(0, 0, True)

        def cpair1(t, carry):
            chunk1(2 * t, 0)
            chunk1(2 * t + 1, 1)
            return carry

        lax.fori_loop(0, _NCH // 2, cpair1, 0)
        chunk1(_NCH - 1, 0)

        plsc.subcore_barrier()
        pltpu.sync_copy(acc.at[pl.ds(sid * _RPS, _RPS)],
                        zout_hbm.at[cid, pl.ds(sid * _RPS, _RPS)])
        plsc.subcore_barrier()
        zero_acc()
        plsc.subcore_barrier()

        def edge2(u, j, p):
            ap, _ = ab[p]
            rel_c = rels[u]

            @plsc.parallel_loop(0, _B, unroll=4)
            def body(i):
                rid = rel_c[pl.ds(j * _B + i, 16)][0]
                ev = re_v[rid, :]
                srow = sb[p][i, :]
                for h in range(_H):
                    svec = _xlane(srow, hsplat[h])
                    vh = ap[i, pl.ds(h * _DK, _DK)]
                    wb[p][i, pl.ds(h * _DK, _DK)] = (vh + ev) * svec

        def chunk2(c, u):
            idx_wait(u, False)
            g_issue(v_hbm, 0, 0, ab, u)
            sl_issue(c, 0, 0)
            d_issue(c, 0, 0)

            @pl.when(c + 1 < _NCH)
            def _():
                idx_issue(c + 1, 1 - u, False)

            def pair(t, pcarry):
                for half in range(2):
                    p = half
                    j = 2 * t + half
                    nj = j + 1

                    @pl.when(j >= 1)
                    def _():
                        sc_wait(1 - p)

                    @pl.when(nj < _CB)
                    def _():
                        g_issue(v_hbm, nj, 1 - p, ab, u)
                        sl_issue(c, nj, 1 - p)
                        d_issue(c, nj, 1 - p)

                    g_wait(v_hbm, p, ab)
                    sl_wait(p)
                    d_wait(p)

                    edge2(u, j, p)
                    sc_issue(p)
                return pcarry

            lax.fori_loop(0, _CB // 2, pair, 0)
            sc_wait(1)

        idx_issue(0, 0, False)

        def cpair2(t, carry):
            chunk2(2 * t, 0)
            chunk2(2 * t + 1, 1)
            return carry

        lax.fori_loop(0, _NCH // 2, cpair2, 0)
        chunk2(_NCH - 1, 0)

        plsc.subcore_barrier()
        pltpu.sync_copy(acc.at[pl.ds(sid * _RPS, _RPS)],
                        mout_hbm.at[cid, pl.ds(sid * _RPS, _RPS)])

    return ek(q, k, v, src, dst1, rel, re)


def kernel(x, edge_index, rel_ids, rel_embed, layers):
    src = edge_index[0].astype(jnp.int32)
    dst1 = edge_index[1].astype(jnp.int32)
    rel = rel_ids.astype(jnp.int32)
    out = x
    for p in layers:
        q, k, v = _qkv(out, p["Wq"], p["bq"], p["Wk"], p["Wv"])
        zp, mp, _ = _edge_sc(q, k, v, src, dst1, rel, rel_embed)
        out = _post(out, mp[0, :_N], mp[1, :_N], zp[0, :_N], zp[1, :_N], p)
    return out
